# Initial kernel scaffold; baseline (speedup 1.0000x reference)
import jax
import jax.numpy as jnp
from jax.experimental import pallas as pl


def kernel(features, edge_index, W1, b1, W2, b2):
    raise NotImplementedError("write your pallas kernel here")


# speedup vs baseline: 24.0810x; 24.0810x over previous
import functools

import jax
import jax.numpy as jnp
from jax import lax
from jax.experimental import pallas as pl
from jax.experimental.pallas import tpu as pltpu
from jax.experimental.pallas import tpu_sc as plsc

NC = 2
NS = 16
LANES = 16
IDXW = 128
CHUNK = 8
TB = 1024


def _make_deg(NP, R):
    K2 = R // NS
    nchunks = K2 // CHUNK
    ZT = NP // NS
    mesh = plsc.VectorSubcoreMesh(core_axis_name="c", subcore_axis_name="s")

    @functools.partial(
        pl.kernel,
        out_type=jax.ShapeDtypeStruct((NC, NP), jnp.float32),
        mesh=mesh,
        compiler_params=pltpu.CompilerParams(use_tc_tiling_on_sc=False),
        scratch_types=[
            pltpu.VMEM_SHARED((NP,), jnp.float32),
            pltpu.VMEM((CHUNK, IDXW), jnp.int32),
            pltpu.VMEM((IDXW,), jnp.float32),
            pltpu.VMEM((ZT,), jnp.float32),
            pltpu.SemaphoreType.DMA,
        ],
    )
    def deg(idx_hbm, out_hbm, acc, sidx, ones, zbuf, sem):
        c = lax.axis_index("c")
        s = lax.axis_index("s")

        def fill_ones(i, carry):
            ones[pl.ds(i * LANES, LANES)] = jnp.ones((LANES,), jnp.float32)
            return carry

        lax.fori_loop(0, IDXW // LANES, fill_ones, 0)

        def fill_z(i, carry):
            zbuf[pl.ds(i * LANES, LANES)] = jnp.zeros((LANES,), jnp.float32)
            return carry

        lax.fori_loop(0, ZT // LANES, fill_z, 0)
        pltpu.sync_copy(zbuf, acc.at[pl.ds(s * ZT, ZT)])
        plsc.subcore_barrier()

        def chunk(i, carry):
            r0 = s * K2 + i * CHUNK
            pltpu.sync_copy(idx_hbm.at[c, pl.ds(r0, CHUNK), :], sidx)
            cps = [
                pltpu.async_copy(ones, acc.at[sidx.at[j]], sem, add=True)
                for j in range(CHUNK)
            ]
            for cp in cps:
                cp.wait()
            return carry

        lax.fori_loop(0, nchunks, chunk, 0)
        plsc.subcore_barrier()
        pltpu.sync_copy(acc.at[pl.ds(s * ZT, ZT)], out_hbm.at[c, pl.ds(s * ZT, ZT)])

    return deg


def _make_agg(NP, R, K, D):
    nchunks = K // CHUNK
    ZT = NP // NS
    ZB = 112
    assert ZT % ZB == 0
    mesh = plsc.VectorSubcoreMesh(core_axis_name="c", subcore_axis_name="s")

    @functools.partial(
        pl.kernel,
        out_type=jax.ShapeDtypeStruct((NC, NP, D), jnp.float32),
        mesh=mesh,
        compiler_params=pltpu.CompilerParams(use_tc_tiling_on_sc=False),
        scratch_types=[
            pltpu.VMEM_SHARED((NP, D), jnp.float32),
            pltpu.VMEM((CHUNK, IDXW), jnp.int32),
            pltpu.VMEM((CHUNK, IDXW), jnp.int32),
            pltpu.VMEM((CHUNK, IDXW, D), jnp.float32),
            pltpu.VMEM((ZB, D), jnp.float32),
            pltpu.SemaphoreType.DMA,
            pltpu.SemaphoreType.DMA,
        ],
    )
    def agg(idx_hbm, table_hbm, out_hbm, acc, sidx, didx, rows, zbuf, gsem, ssem):
        c = lax.axis_index("c")
        s = lax.axis_index("s")
        wid = c * NS + s

        def fill_z(i, carry):
            zbuf[i, :] = jnp.zeros((LANES,), jnp.float32)
            return carry

        lax.fori_loop(0, ZB, fill_z, 0)

        def zcp(t, carry):
            pltpu.sync_copy(zbuf, acc.at[pl.ds(s * ZT + t * ZB, ZB), :])
            return carry

        lax.fori_loop(0, ZT // ZB, zcp, 0)
        plsc.subcore_barrier()

        def chunk(i, carry):
            r0 = wid * K + i * CHUNK
            pltpu.sync_copy(idx_hbm.at[0, pl.ds(r0, CHUNK), :], sidx)
            pltpu.sync_copy(idx_hbm.at[1, pl.ds(r0, CHUNK), :], didx)
            g = [
                pltpu.async_copy(table_hbm.at[sidx.at[j]], rows.at[j], gsem)
                for j in range(CHUNK)
            ]
            for cp in g:
                cp.wait()
            sc = [
                pltpu.async_copy(rows.at[j], acc.at[didx.at[j]], ssem, add=True)
                for j in range(CHUNK)
            ]
            for cp in sc:
                cp.wait()
            return carry

        lax.fori_loop(0, nchunks, chunk, 0)
        plsc.subcore_barrier()
        pltpu.sync_copy(
            acc.at[pl.ds(s * ZT, ZT), :], out_hbm.at[c, pl.ds(s * ZT, ZT), :]
        )

    return agg


def _prep(cs, cd, featsp, W1):
    NP, DIN = featsp.shape
    DH = W1.shape[1]
    G = NP // TB

    def body(cs_ref, cd_ref, x_ref, w_ref, t_ref, ns_ref, nd_ref):
        ns = lax.rsqrt(jnp.maximum(cs_ref[:], 1.0))
        nd = lax.rsqrt(jnp.maximum(cd_ref[:], 1.0))
        ns_ref[:] = ns
        nd_ref[:] = nd
        t_ref[:] = jnp.dot(
            x_ref[:] * ns[:, None], w_ref[:], preferred_element_type=jnp.float32
        )

    return pl.pallas_call(
        body,
        grid=(G,),
        in_specs=[
            pl.BlockSpec((TB,), lambda i: (i,)),
            pl.BlockSpec((TB,), lambda i: (i,)),
            pl.BlockSpec((TB, DIN), lambda i: (i, 0)),
            pl.BlockSpec((DIN, DH), lambda i: (0, 0)),
        ],
        out_specs=[
            pl.BlockSpec((TB, DH), lambda i: (i, 0)),
            pl.BlockSpec((TB,), lambda i: (i,)),
            pl.BlockSpec((TB,), lambda i: (i,)),
        ],
        out_shape=[
            jax.ShapeDtypeStruct((NP, DH), jnp.float32),
            jax.ShapeDtypeStruct((NP,), jnp.float32),
            jax.ShapeDtypeStruct((NP,), jnp.float32),
        ],
    )(cs, cd, featsp, W1)


def _mid(part1, ns, nd, b1):
    _, NP, DH = part1.shape
    G = NP // TB

    def body(m_ref, ns_ref, nd_ref, b_ref, t_ref):
        m = m_ref[0] + m_ref[1]
        x1 = jnp.maximum(m * nd_ref[:][:, None] + b_ref[:][None, :], 0.0)
        t_ref[:] = x1 * ns_ref[:][:, None]

    return pl.pallas_call(
        body,
        grid=(G,),
        in_specs=[
            pl.BlockSpec((NC, TB, DH), lambda i: (0, i, 0)),
            pl.BlockSpec((TB,), lambda i: (i,)),
            pl.BlockSpec((TB,), lambda i: (i,)),
            pl.BlockSpec((DH,), lambda i: (0,)),
        ],
        out_specs=pl.BlockSpec((TB, DH), lambda i: (i, 0)),
        out_shape=jax.ShapeDtypeStruct((NP, DH), jnp.float32),
    )(part1, ns, nd, b1)


def _fin(part2, nd, W2, b2):
    _, NP, DH = part2.shape
    DO = W2.shape[1]
    G = NP // TB

    def body(m_ref, nd_ref, w_ref, b_ref, o_ref):
        m = m_ref[0] + m_ref[1]
        y = jnp.dot(m, w_ref[:], preferred_element_type=jnp.float32)
        o_ref[:] = y * nd_ref[:][:, None] + b_ref[:][None, :]

    return pl.pallas_call(
        body,
        grid=(G,),
        in_specs=[
            pl.BlockSpec((NC, TB, DH), lambda i: (0, i, 0)),
            pl.BlockSpec((TB,), lambda i: (i,)),
            pl.BlockSpec((DH, DO), lambda i: (0, 0)),
            pl.BlockSpec((DO,), lambda i: (0,)),
        ],
        out_specs=pl.BlockSpec((TB, DO), lambda i: (i, 0)),
        out_shape=jax.ShapeDtypeStruct((NP, DO), jnp.float32),
    )(part2, nd, W2, b2)


def kernel(features, edge_index, W1, b1, W2, b2):
    N, DIN = features.shape
    E = edge_index.shape[1]
    DH = W1.shape[1]

    rows = -(-E // IDXW)
    K = -(-rows // (NC * NS * CHUNK)) * CHUNK
    R = K * NC * NS
    Epad = R * IDXW
    NP = -(-(N + 1) // TB) * TB

    idxp = jnp.concatenate(
        [edge_index, jnp.full((2, Epad - E), N, jnp.int32)], axis=1
    )
    idx2 = idxp.reshape(2, R, IDXW)
    featsp = jnp.concatenate(
        [features, jnp.zeros((NP - N, DIN), features.dtype)], axis=0
    )

    cnts = _make_deg(NP, R)(idx2)
    table1, ns, nd = _prep(cnts[0], cnts[1], featsp, W1)
    part1 = _make_agg(NP, R, K, DH)(idx2, table1)
    table2 = _mid(part1, ns, nd, b1)
    part2 = _make_agg(NP, R, K, DH)(idx2, table2)
    yfull = _fin(part2, nd, W2, b2)
    return yfull[:N]


# speedup vs baseline: 27.6606x; 1.1487x over previous
import functools

import jax
import jax.numpy as jnp
from jax import lax
from jax.experimental import pallas as pl
from jax.experimental.pallas import tpu as pltpu
from jax.experimental.pallas import tpu_sc as plsc

NC = 2
NS = 16
LANES = 16
IDXW = 128
CHUNK = 8
TB = 1024


def _make_deg(NP, R):
    K2 = R // NS
    DCH = 8
    NB = 2
    nchunks = K2 // DCH
    G = nchunks // NB
    assert nchunks % NB == 0
    ZT = NP // NS
    mesh = plsc.VectorSubcoreMesh(core_axis_name="c", subcore_axis_name="s")

    @functools.partial(
        pl.kernel,
        out_type=jax.ShapeDtypeStruct((NC, NP), jnp.float32),
        mesh=mesh,
        compiler_params=pltpu.CompilerParams(use_tc_tiling_on_sc=False),
        scratch_types=[
            pltpu.VMEM_SHARED((NP,), jnp.float32),
            pltpu.VMEM((NB, DCH, IDXW), jnp.int32),
            pltpu.VMEM((IDXW,), jnp.float32),
            pltpu.VMEM((ZT,), jnp.float32),
            pltpu.SemaphoreType.DMA,
            pltpu.SemaphoreType.DMA,
        ],
    )
    def deg(idx_hbm, out_hbm, acc, sidx, ones, zbuf, sem0, sem1):
        c = lax.axis_index("c")
        s = lax.axis_index("s")
        sems = [sem0, sem1]

        def fill_ones(i, carry):
            ones[pl.ds(i * LANES, LANES)] = jnp.ones((LANES,), jnp.float32)
            return carry

        lax.fori_loop(0, IDXW // LANES, fill_ones, 0)

        def fill_z(i, carry):
            zbuf[pl.ds(i * LANES, LANES)] = jnp.zeros((LANES,), jnp.float32)
            return carry

        lax.fori_loop(0, ZT // LANES, fill_z, 0)
        pltpu.sync_copy(zbuf, acc.at[pl.ds(s * ZT, ZT)])
        plsc.subcore_barrier()

        base = s * K2
        for b in range(NB):
            pltpu.sync_copy(
                idx_hbm.at[c, pl.ds(base + b * DCH, DCH), :], sidx.at[b]
            )

        def group(g, carry):
            cps = []
            for b in range(NB):
                cps.append([
                    pltpu.async_copy(
                        ones, acc.at[sidx.at[b, j]], sems[b], add=True
                    )
                    for j in range(DCH)
                ])
            for b in range(NB):
                for cp in cps[b]:
                    cp.wait()
                nxt = jnp.minimum((g + 1) * NB + b, nchunks - 1)
                pltpu.sync_copy(
                    idx_hbm.at[c, pl.ds(base + nxt * DCH, DCH), :], sidx.at[b]
                )
            return carry

        lax.fori_loop(0, G, group, 0)
        plsc.subcore_barrier()
        pltpu.sync_copy(acc.at[pl.ds(s * ZT, ZT)], out_hbm.at[c, pl.ds(s * ZT, ZT)])

    return deg


def _make_agg(NP, R, K, D):
    CH = 4
    NB = 2
    nchunks = K // CH
    G = nchunks // NB
    assert nchunks % NB == 0
    ZT = NP // NS
    ZB = 112
    assert ZT % ZB == 0
    mesh = plsc.VectorSubcoreMesh(core_axis_name="c", subcore_axis_name="s")

    @functools.partial(
        pl.kernel,
        out_type=jax.ShapeDtypeStruct((NC, NP, D), jnp.float32),
        mesh=mesh,
        compiler_params=pltpu.CompilerParams(use_tc_tiling_on_sc=False),
        scratch_types=[
            pltpu.VMEM_SHARED((NP, D), jnp.float32),
            pltpu.VMEM((NB, 2, CH, IDXW), jnp.int32),
            pltpu.VMEM((NB, CH, IDXW, D), jnp.float32),
            pltpu.VMEM((ZB, D), jnp.float32),
            pltpu.SemaphoreType.DMA,
            pltpu.SemaphoreType.DMA,
            pltpu.SemaphoreType.DMA,
            pltpu.SemaphoreType.DMA,
        ],
    )
    def agg(idx_hbm, table_hbm, out_hbm, acc, idxb, rows, zbuf,
            gsem0, gsem1, ssem0, ssem1):
        c = lax.axis_index("c")
        s = lax.axis_index("s")
        wid = c * NS + s
        gsems = [gsem0, gsem1]
        ssems = [ssem0, ssem1]

        def fill_z(i, carry):
            zbuf[i, :] = jnp.zeros((LANES,), jnp.float32)
            return carry

        lax.fori_loop(0, ZB, fill_z, 0)

        def zcp(t, carry):
            pltpu.sync_copy(zbuf, acc.at[pl.ds(s * ZT + t * ZB, ZB), :])
            return carry

        lax.fori_loop(0, ZT // ZB, zcp, 0)
        plsc.subcore_barrier()

        base = wid * K

        def stage(b, ci):
            pltpu.sync_copy(
                idx_hbm.at[:, pl.ds(base + ci * CH, CH), :], idxb.at[b]
            )
            return [
                pltpu.async_copy(
                    table_hbm.at[idxb.at[b, 0, j]], rows.at[b, j], gsems[b]
                )
                for j in range(CH)
            ]

        gcps = [stage(b, b) for b in range(NB)]

        def group(g, carry):
            scps = []
            for b in range(NB):
                for cp in gcps[b]:
                    cp.wait()
                scps.append([
                    pltpu.async_copy(
                        rows.at[b, j], acc.at[idxb.at[b, 1, j]], ssems[b],
                        add=True,
                    )
                    for j in range(CH)
                ])
            for b in range(NB):
                for cp in scps[b]:
                    cp.wait()
                nxt = jnp.minimum((g + 1) * NB + b, nchunks - 1)
                gcps[b] = stage(b, nxt)
            return carry

        lax.fori_loop(0, G, group, 0)
        for b in range(NB):
            for cp in gcps[b]:
                cp.wait()
        plsc.subcore_barrier()
        pltpu.sync_copy(
            acc.at[pl.ds(s * ZT, ZT), :], out_hbm.at[c, pl.ds(s * ZT, ZT), :]
        )

    return agg


def _prep(cs, cd, featsp, W1):
    NP, DIN = featsp.shape
    DH = W1.shape[1]
    G = NP // TB

    def body(cs_ref, cd_ref, x_ref, w_ref, t_ref, ns_ref, nd_ref):
        ns = lax.rsqrt(jnp.maximum(cs_ref[:], 1.0))
        nd = lax.rsqrt(jnp.maximum(cd_ref[:], 1.0))
        ns_ref[:] = ns
        nd_ref[:] = nd
        t_ref[:] = jnp.dot(
            x_ref[:] * ns[:, None], w_ref[:], preferred_element_type=jnp.float32
        )

    return pl.pallas_call(
        body,
        grid=(G,),
        in_specs=[
            pl.BlockSpec((TB,), lambda i: (i,)),
            pl.BlockSpec((TB,), lambda i: (i,)),
            pl.BlockSpec((TB, DIN), lambda i: (i, 0)),
            pl.BlockSpec((DIN, DH), lambda i: (0, 0)),
        ],
        out_specs=[
            pl.BlockSpec((TB, DH), lambda i: (i, 0)),
            pl.BlockSpec((TB,), lambda i: (i,)),
            pl.BlockSpec((TB,), lambda i: (i,)),
        ],
        out_shape=[
            jax.ShapeDtypeStruct((NP, DH), jnp.float32),
            jax.ShapeDtypeStruct((NP,), jnp.float32),
            jax.ShapeDtypeStruct((NP,), jnp.float32),
        ],
    )(cs, cd, featsp, W1)


def _mid(part1, ns, nd, b1):
    _, NP, DH = part1.shape
    G = NP // TB

    def body(m_ref, ns_ref, nd_ref, b_ref, t_ref):
        m = m_ref[0] + m_ref[1]
        x1 = jnp.maximum(m * nd_ref[:][:, None] + b_ref[:][None, :], 0.0)
        t_ref[:] = x1 * ns_ref[:][:, None]

    return pl.pallas_call(
        body,
        grid=(G,),
        in_specs=[
            pl.BlockSpec((NC, TB, DH), lambda i: (0, i, 0)),
            pl.BlockSpec((TB,), lambda i: (i,)),
            pl.BlockSpec((TB,), lambda i: (i,)),
            pl.BlockSpec((DH,), lambda i: (0,)),
        ],
        out_specs=pl.BlockSpec((TB, DH), lambda i: (i, 0)),
        out_shape=jax.ShapeDtypeStruct((NP, DH), jnp.float32),
    )(part1, ns, nd, b1)


def _fin(part2, nd, W2, b2):
    _, NP, DH = part2.shape
    DO = W2.shape[1]
    G = NP // TB

    def body(m_ref, nd_ref, w_ref, b_ref, o_ref):
        m = m_ref[0] + m_ref[1]
        y = jnp.dot(m, w_ref[:], preferred_element_type=jnp.float32)
        o_ref[:] = y * nd_ref[:][:, None] + b_ref[:][None, :]

    return pl.pallas_call(
        body,
        grid=(G,),
        in_specs=[
            pl.BlockSpec((NC, TB, DH), lambda i: (0, i, 0)),
            pl.BlockSpec((TB,), lambda i: (i,)),
            pl.BlockSpec((DH, DO), lambda i: (0, 0)),
            pl.BlockSpec((DO,), lambda i: (0,)),
        ],
        out_specs=pl.BlockSpec((TB, DO), lambda i: (i, 0)),
        out_shape=jax.ShapeDtypeStruct((NP, DO), jnp.float32),
    )(part2, nd, W2, b2)


def kernel(features, edge_index, W1, b1, W2, b2):
    N, DIN = features.shape
    E = edge_index.shape[1]
    DH = W1.shape[1]

    rows = -(-E // IDXW)
    K = -(-rows // (NC * NS * CHUNK)) * CHUNK
    R = K * NC * NS
    Epad = R * IDXW
    NP = -(-(N + 1) // TB) * TB

    idxp = jnp.concatenate(
        [edge_index, jnp.full((2, Epad - E), N, jnp.int32)], axis=1
    )
    idx2 = idxp.reshape(2, R, IDXW)
    featsp = jnp.concatenate(
        [features, jnp.zeros((NP - N, DIN), features.dtype)], axis=0
    )

    cnts = _make_deg(NP, R)(idx2)
    table1, ns, nd = _prep(cnts[0], cnts[1], featsp, W1)
    part1 = _make_agg(NP, R, K, DH)(idx2, table1)
    table2 = _mid(part1, ns, nd, b1)
    part2 = _make_agg(NP, R, K, DH)(idx2, table2)
    yfull = _fin(part2, nd, W2, b2)
    return yfull[:N]


# speedup vs baseline: 36.2354x; 1.3100x over previous
import functools

import jax
import jax.numpy as jnp
from jax import lax
from jax.experimental import pallas as pl
from jax.experimental.pallas import tpu as pltpu
from jax.experimental.pallas import tpu_sc as plsc

NC = 2
NS = 16
LANES = 16
IDXW = 128
CHUNK = 8
TB = 7168


def _make_deg(NP, R):
    K2 = R // NS
    DCH = 16
    NB = 2
    nchunks = K2 // DCH
    G = nchunks // NB
    assert nchunks % NB == 0
    ZT = NP // NS
    mesh = plsc.VectorSubcoreMesh(core_axis_name="c", subcore_axis_name="s")

    @functools.partial(
        pl.kernel,
        out_type=jax.ShapeDtypeStruct((NC, NP), jnp.float32),
        mesh=mesh,
        compiler_params=pltpu.CompilerParams(use_tc_tiling_on_sc=False),
        scratch_types=[
            pltpu.VMEM_SHARED((NP,), jnp.float32),
            pltpu.VMEM((NB, DCH, IDXW), jnp.int32),
            pltpu.VMEM((IDXW,), jnp.float32),
            pltpu.VMEM((ZT,), jnp.float32),
            pltpu.SemaphoreType.DMA,
            pltpu.SemaphoreType.DMA,
            pltpu.SemaphoreType.DMA,
            pltpu.SemaphoreType.DMA,
        ],
    )
    def deg(idx_hbm, out_hbm, acc, sidx, ones, zbuf, sem0, sem1, isem0, isem1):
        c = lax.axis_index("c")
        s = lax.axis_index("s")
        sems = [sem0, sem1]
        isems = [isem0, isem1]

        def fill_ones(i, carry):
            ones[pl.ds(i * LANES, LANES)] = jnp.ones((LANES,), jnp.float32)
            return carry

        lax.fori_loop(0, IDXW // LANES, fill_ones, 0)

        def fill_z(i, carry):
            zbuf[pl.ds(i * LANES, LANES)] = jnp.zeros((LANES,), jnp.float32)
            return carry

        lax.fori_loop(0, ZT // LANES, fill_z, 0)
        pltpu.sync_copy(zbuf, acc.at[pl.ds(s * ZT, ZT)])
        plsc.subcore_barrier()

        base = s * K2
        icps = [
            pltpu.async_copy(
                idx_hbm.at[c, pl.ds(base + b * DCH, DCH), :], sidx.at[b],
                isems[b],
            )
            for b in range(NB)
        ]

        def group(g, carry):
            cps = []
            for b in range(NB):
                icps[b].wait()
                cps.append([
                    pltpu.async_copy(
                        ones, acc.at[sidx.at[b, j]], sems[b], add=True
                    )
                    for j in range(DCH)
                ])
            for b in range(NB):
                for cp in cps[b]:
                    cp.wait()
                nxt = jnp.minimum((g + 1) * NB + b, nchunks - 1)
                icps[b] = pltpu.async_copy(
                    idx_hbm.at[c, pl.ds(base + nxt * DCH, DCH), :], sidx.at[b],
                    isems[b],
                )
            return carry

        lax.fori_loop(0, G, group, 0)
        for b in range(NB):
            pltpu.make_async_copy(
                idx_hbm.at[c, pl.ds(0, DCH), :], sidx.at[b], isems[b]
            ).wait()
        plsc.subcore_barrier()
        pltpu.sync_copy(acc.at[pl.ds(s * ZT, ZT)], out_hbm.at[c, pl.ds(s * ZT, ZT)])

    return deg


def _make_agg(NP, R, K, D, CH, NB):
    nchunks = K // CH
    G = nchunks // NB
    assert nchunks % NB == 0
    ZT = NP // NS
    mesh = plsc.VectorSubcoreMesh(core_axis_name="c", subcore_axis_name="s")

    @functools.partial(
        pl.kernel,
        out_type=jax.ShapeDtypeStruct((NC, NP, D), jnp.float32),
        mesh=mesh,
        compiler_params=pltpu.CompilerParams(use_tc_tiling_on_sc=False),
        scratch_types=[
            pltpu.VMEM_SHARED((NP, D), jnp.float32),
            pltpu.VMEM((NB, 2, CH, IDXW), jnp.int32),
            pltpu.VMEM((NB, CH, IDXW, D), jnp.float32),
            pltpu.SemaphoreType.DMA,
            pltpu.SemaphoreType.DMA,
            pltpu.SemaphoreType.DMA,
            pltpu.SemaphoreType.DMA,
            pltpu.SemaphoreType.DMA,
            pltpu.SemaphoreType.DMA,
        ],
    )
    def agg(idx_hbm, table_hbm, zeros_hbm, out_hbm, acc, idxb, rows,
            gsem0, gsem1, ssem0, ssem1, isem0, isem1):
        c = lax.axis_index("c")
        s = lax.axis_index("s")
        wid = c * NS + s
        gsems = [gsem0, gsem1]
        ssems = [ssem0, ssem1]
        isems = [isem0, isem1]

        pltpu.sync_copy(zeros_hbm.at[pl.ds(s * ZT, ZT), :],
                        acc.at[pl.ds(s * ZT, ZT), :])
        plsc.subcore_barrier()

        base = wid * K

        def fire_idx(b, ci):
            return pltpu.async_copy(
                idx_hbm.at[:, pl.ds(base + ci * CH, CH), :], idxb.at[b],
                isems[b],
            )

        def fire_gathers(b):
            return [
                pltpu.async_copy(
                    table_hbm.at[idxb.at[b, 0, j]], rows.at[b, j], gsems[b]
                )
                for j in range(CH)
            ]

        icps = [fire_idx(b, b) for b in range(NB)]
        gcps = []
        for b in range(NB):
            icps[b].wait()
            gcps.append(fire_gathers(b))

        def group(g, carry):
            scps = []
            for b in range(NB):
                for cp in gcps[b]:
                    cp.wait()
                scps.append([
                    pltpu.async_copy(
                        rows.at[b, j], acc.at[idxb.at[b, 1, j]], ssems[b],
                        add=True,
                    )
                    for j in range(CH)
                ])
            for b in range(NB):
                for cp in scps[b]:
                    cp.wait()
                nxt = jnp.minimum((g + 1) * NB + b, nchunks - 1)
                icps[b] = fire_idx(b, nxt)
            for b in range(NB):
                icps[b].wait()
                gcps[b] = fire_gathers(b)
            return carry

        lax.fori_loop(0, G, group, 0)
        for b in range(NB):
            for j in range(CH):
                pltpu.make_async_copy(
                    table_hbm.at[idxb.at[b, 0, j]], rows.at[b, j], gsems[b]
                ).wait()
        plsc.subcore_barrier()
        pltpu.sync_copy(
            acc.at[pl.ds(s * ZT, ZT), :], out_hbm.at[c, pl.ds(s * ZT, ZT), :]
        )

    return agg


def _prep(cs, featsp):
    NP, DW = featsp.shape
    G = NP // TB

    def body(cs_ref, x_ref, t_ref):
        ns = lax.rsqrt(jnp.maximum(cs_ref[:], 1.0))
        t_ref[:] = x_ref[:] * ns[:, None]

    return pl.pallas_call(
        body,
        grid=(G,),
        in_specs=[
            pl.BlockSpec((TB,), lambda i: (i,)),
            pl.BlockSpec((TB, DW), lambda i: (i, 0)),
        ],
        out_specs=pl.BlockSpec((TB, DW), lambda i: (i, 0)),
        out_shape=jax.ShapeDtypeStruct((NP, DW), jnp.float32),
    )(cs, featsp)


def _mid(part1, cs, cd, W1p, b1, W2):
    _, NP, DW = part1.shape
    DO = W2.shape[1]
    G = NP // TB

    def body(m_ref, cs_ref, cd_ref, w1_ref, b_ref, w2_ref, t_ref):
        ns = lax.rsqrt(jnp.maximum(cs_ref[:], 1.0))
        nd = lax.rsqrt(jnp.maximum(cd_ref[:], 1.0))
        m = m_ref[0] + m_ref[1]
        h = jnp.dot(m, w1_ref[:], preferred_element_type=jnp.float32)
        x1 = jnp.maximum(h * nd[:, None] + b_ref[:][None, :], 0.0)
        y = jnp.dot(
            x1 * ns[:, None], w2_ref[:], preferred_element_type=jnp.float32
        )
        t_ref[:] = jnp.concatenate(
            [y, jnp.zeros((y.shape[0], 8 - DO), jnp.float32)], axis=1
        )

    return pl.pallas_call(
        body,
        grid=(G,),
        in_specs=[
            pl.BlockSpec((NC, TB, DW), lambda i: (0, i, 0)),
            pl.BlockSpec((TB,), lambda i: (i,)),
            pl.BlockSpec((TB,), lambda i: (i,)),
            pl.BlockSpec((DW, DW), lambda i: (0, 0)),
            pl.BlockSpec((DW,), lambda i: (0,)),
            pl.BlockSpec((DW, DO), lambda i: (0, 0)),
        ],
        out_specs=pl.BlockSpec((TB, 8), lambda i: (i, 0)),
        out_shape=jax.ShapeDtypeStruct((NP, 8), jnp.float32),
    )(part1, cs, cd, W1p, b1, W2)


def _fin(part2, cd, b2, N):
    _, NP, DP = part2.shape
    DO = b2.shape[0]
    G = -(-N // TB)

    def body(m_ref, cd_ref, b_ref, o_ref):
        nd = lax.rsqrt(jnp.maximum(cd_ref[:], 1.0))
        m = m_ref[0, :, :DO] + m_ref[1, :, :DO]
        o_ref[:] = m * nd[:, None] + b_ref[:][None, :]

    return pl.pallas_call(
        body,
        grid=(G,),
        in_specs=[
            pl.BlockSpec((NC, TB, DP), lambda i: (0, i, 0)),
            pl.BlockSpec((TB,), lambda i: (i,)),
            pl.BlockSpec((DO,), lambda i: (0,)),
        ],
        out_specs=pl.BlockSpec((TB, DO), lambda i: (i, 0)),
        out_shape=jax.ShapeDtypeStruct((N, DO), jnp.float32),
    )(part2, cd, b2)


def kernel(features, edge_index, W1, b1, W2, b2):
    N, DIN = features.shape
    E = edge_index.shape[1]
    DH = W1.shape[1]
    DO = W2.shape[1]

    rows = -(-E // IDXW)
    K = -(-rows // (NC * NS * CHUNK)) * CHUNK
    R = K * NC * NS
    Epad = R * IDXW
    NP = -(-(N + 1) // TB) * TB
    DW = 16

    idxp = jnp.concatenate(
        [edge_index, jnp.full((2, Epad - E), N, jnp.int32)], axis=1
    )
    idx2 = idxp.reshape(2, R, IDXW)
    featsp = jnp.zeros((NP, DW), features.dtype).at[:N, :DIN].set(features)
    W1p = jnp.zeros((DW, DH), W1.dtype).at[:DIN, :].set(W1)

    cnts = _make_deg(NP, R)(idx2)
    xs = _prep(cnts[0], featsp)
    part1 = _make_agg(NP, R, K, DW, 4, 2)(idx2, xs, jnp.zeros((NP, DW), jnp.float32))
    table2 = _mid(part1, cnts[0], cnts[1], W1p, b1, W2)
    part2 = _make_agg(NP, R, K, 8, 8, 2)(idx2, table2, jnp.zeros((NP, 8), jnp.float32))
    return _fin(part2, cnts[1], b2, N)


# speedup vs baseline: 36.5586x; 1.0089x over previous
import functools

import jax
import jax.numpy as jnp
from jax import lax
from jax.experimental import pallas as pl
from jax.experimental.pallas import tpu as pltpu
from jax.experimental.pallas import tpu_sc as plsc

NC = 2
NS = 16
LANES = 16
IDXW = 128
CHUNK = 8
TB = 7168


def _make_deg(NP, R):
    K2 = R // NS
    DCH = 12
    NB = 2
    nchunks = K2 // DCH
    G = nchunks // NB
    assert nchunks % NB == 0
    ZT = NP // NS
    mesh = plsc.VectorSubcoreMesh(core_axis_name="c", subcore_axis_name="s")

    @functools.partial(
        pl.kernel,
        out_type=jax.ShapeDtypeStruct((NC, NP), jnp.float32),
        mesh=mesh,
        compiler_params=pltpu.CompilerParams(use_tc_tiling_on_sc=False),
        scratch_types=[
            pltpu.VMEM_SHARED((NP,), jnp.float32),
            pltpu.VMEM((NB, DCH, IDXW), jnp.int32),
            pltpu.VMEM((IDXW,), jnp.float32),
            pltpu.VMEM((ZT,), jnp.float32),
            pltpu.SemaphoreType.DMA,
            pltpu.SemaphoreType.DMA,
            pltpu.SemaphoreType.DMA,
            pltpu.SemaphoreType.DMA,
        ],
    )
    def deg(idx_hbm, out_hbm, acc, sidx, ones, zbuf, sem0, sem1, isem0, isem1):
        c = lax.axis_index("c")
        s = lax.axis_index("s")
        sems = [sem0, sem1]
        isems = [isem0, isem1]

        def fill_ones(i, carry):
            ones[pl.ds(i * LANES, LANES)] = jnp.ones((LANES,), jnp.float32)
            return carry

        lax.fori_loop(0, IDXW // LANES, fill_ones, 0)

        def fill_z(i, carry):
            zbuf[pl.ds(i * LANES, LANES)] = jnp.zeros((LANES,), jnp.float32)
            return carry

        lax.fori_loop(0, ZT // LANES, fill_z, 0)
        pltpu.sync_copy(zbuf, acc.at[pl.ds(s * ZT, ZT)])
        plsc.subcore_barrier()

        base = s * K2
        icps = [
            pltpu.async_copy(
                idx_hbm.at[c, pl.ds(base + b * DCH, DCH), :], sidx.at[b],
                isems[b],
            )
            for b in range(NB)
        ]

        def group(g, carry):
            cps = []
            for b in range(NB):
                icps[b].wait()
                cps.append([
                    pltpu.async_copy(
                        ones, acc.at[sidx.at[b, j]], sems[b], add=True
                    )
                    for j in range(DCH)
                ])
            for b in range(NB):
                for cp in cps[b]:
                    cp.wait()
                nxt = jnp.minimum((g + 1) * NB + b, nchunks - 1)
                icps[b] = pltpu.async_copy(
                    idx_hbm.at[c, pl.ds(base + nxt * DCH, DCH), :], sidx.at[b],
                    isems[b],
                )
            return carry

        lax.fori_loop(0, G, group, 0)
        for b in range(NB):
            pltpu.make_async_copy(
                idx_hbm.at[c, pl.ds(0, DCH), :], sidx.at[b], isems[b]
            ).wait()
        plsc.subcore_barrier()
        pltpu.sync_copy(acc.at[pl.ds(s * ZT, ZT)], out_hbm.at[c, pl.ds(s * ZT, ZT)])

    return deg


def _make_agg(NP, R, K, D, CH, NB):
    nchunks = K // CH
    G = nchunks // NB
    assert nchunks % NB == 0
    ZT = NP // NS
    mesh = plsc.VectorSubcoreMesh(core_axis_name="c", subcore_axis_name="s")

    @functools.partial(
        pl.kernel,
        out_type=jax.ShapeDtypeStruct((NC, NP, D), jnp.float32),
        mesh=mesh,
        compiler_params=pltpu.CompilerParams(use_tc_tiling_on_sc=False),
        scratch_types=[
            pltpu.VMEM_SHARED((NP, D), jnp.float32),
            pltpu.VMEM((NB, 2, CH, IDXW), jnp.int32),
            pltpu.VMEM((NB, CH, IDXW, D), jnp.float32),
            pltpu.SemaphoreType.DMA,
            pltpu.SemaphoreType.DMA,
            pltpu.SemaphoreType.DMA,
            pltpu.SemaphoreType.DMA,
            pltpu.SemaphoreType.DMA,
            pltpu.SemaphoreType.DMA,
        ],
    )
    def agg(idx_hbm, table_hbm, zeros_hbm, out_hbm, acc, idxb, rows,
            gsem0, gsem1, ssem0, ssem1, isem0, isem1):
        c = lax.axis_index("c")
        s = lax.axis_index("s")
        wid = c * NS + s
        gsems = [gsem0, gsem1]
        ssems = [ssem0, ssem1]
        isems = [isem0, isem1]

        pltpu.sync_copy(zeros_hbm.at[pl.ds(s * ZT, ZT), :],
                        acc.at[pl.ds(s * ZT, ZT), :])
        plsc.subcore_barrier()

        base = wid * K

        def fire_idx(b, ci):
            return pltpu.async_copy(
                idx_hbm.at[:, pl.ds(base + ci * CH, CH), :], idxb.at[b],
                isems[b],
            )

        def fire_gathers(b):
            return [
                pltpu.async_copy(
                    table_hbm.at[idxb.at[b, 0, j]], rows.at[b, j], gsems[b]
                )
                for j in range(CH)
            ]

        icps = [fire_idx(b, b) for b in range(NB)]
        gcps = []
        for b in range(NB):
            icps[b].wait()
            gcps.append(fire_gathers(b))

        def group(g, carry):
            scps = []
            for b in range(NB):
                for cp in gcps[b]:
                    cp.wait()
                scps.append([
                    pltpu.async_copy(
                        rows.at[b, j], acc.at[idxb.at[b, 1, j]], ssems[b],
                        add=True,
                    )
                    for j in range(CH)
                ])
            for b in range(NB):
                for cp in scps[b]:
                    cp.wait()
                nxt = jnp.minimum((g + 1) * NB + b, nchunks - 1)
                icps[b] = fire_idx(b, nxt)
            for b in range(NB):
                icps[b].wait()
                gcps[b] = fire_gathers(b)
            return carry

        lax.fori_loop(0, G, group, 0)
        for b in range(NB):
            for j in range(CH):
                pltpu.make_async_copy(
                    table_hbm.at[idxb.at[b, 0, j]], rows.at[b, j], gsems[b]
                ).wait()
        plsc.subcore_barrier()
        pltpu.sync_copy(
            acc.at[pl.ds(s * ZT, ZT), :], out_hbm.at[c, pl.ds(s * ZT, ZT), :]
        )

    return agg


def _prep(cs, featsp):
    NP, DW = featsp.shape
    G = NP // TB

    def body(cs_ref, x_ref, t_hbm, tv, sem):
        i = pl.program_id(0)
        ns = lax.rsqrt(jnp.maximum(cs_ref[:], 1.0))
        tv[:] = x_ref[:] * ns[:, None]
        cp = pltpu.make_async_copy(tv, t_hbm.at[pl.ds(i * TB, TB), :], sem)
        cp.start()
        cp.wait()

    return pl.pallas_call(
        body,
        grid=(G,),
        in_specs=[
            pl.BlockSpec((TB,), lambda i: (i,)),
            pl.BlockSpec((TB, DW), lambda i: (i, 0)),
        ],
        out_specs=pl.BlockSpec(memory_space=pl.ANY),
        out_shape=jax.ShapeDtypeStruct((NP, DW), jnp.float32),
        scratch_shapes=[
            pltpu.VMEM((TB, DW), jnp.float32),
            pltpu.SemaphoreType.DMA,
        ],
    )(cs, featsp)


def _mid(part1, cs, cd, W1p, b1, W2):
    _, NP, DW = part1.shape
    DO = W2.shape[1]
    G = NP // TB

    def body(m_hbm, cs_ref, cd_ref, w1_ref, b_ref, w2_ref, t_hbm, mv, tv, sem):
        i = pl.program_id(0)
        cin = pltpu.make_async_copy(m_hbm.at[:, pl.ds(i * TB, TB), :], mv, sem)
        cin.start()
        cin.wait()
        ns = lax.rsqrt(jnp.maximum(cs_ref[:], 1.0))
        nd = lax.rsqrt(jnp.maximum(cd_ref[:], 1.0))
        m = mv[0] + mv[1]
        h = jnp.dot(m, w1_ref[:], preferred_element_type=jnp.float32)
        x1 = jnp.maximum(h * nd[:, None] + b_ref[:][None, :], 0.0)
        y = jnp.dot(
            x1 * ns[:, None], w2_ref[:], preferred_element_type=jnp.float32
        )
        tv[:] = jnp.concatenate(
            [y, jnp.zeros((y.shape[0], 8 - DO), jnp.float32)], axis=1
        )
        cout = pltpu.make_async_copy(tv, t_hbm.at[pl.ds(i * TB, TB), :], sem)
        cout.start()
        cout.wait()

    return pl.pallas_call(
        body,
        grid=(G,),
        in_specs=[
            pl.BlockSpec(memory_space=pl.ANY),
            pl.BlockSpec((TB,), lambda i: (i,)),
            pl.BlockSpec((TB,), lambda i: (i,)),
            pl.BlockSpec((DW, DW), lambda i: (0, 0)),
            pl.BlockSpec((DW,), lambda i: (0,)),
            pl.BlockSpec((DW, DO), lambda i: (0, 0)),
        ],
        out_specs=pl.BlockSpec(memory_space=pl.ANY),
        out_shape=jax.ShapeDtypeStruct((NP, 8), jnp.float32),
        scratch_shapes=[
            pltpu.VMEM((NC, TB, DW), jnp.float32),
            pltpu.VMEM((TB, 8), jnp.float32),
            pltpu.SemaphoreType.DMA,
        ],
    )(part1, cs, cd, W1p, b1, W2)


def _fin(part2, cd, b2, N):
    _, NP, DP = part2.shape
    DO = b2.shape[0]
    G = -(-N // TB)

    def body(m_hbm, cd_ref, b_ref, o_ref, mv, sem):
        i = pl.program_id(0)
        cin = pltpu.make_async_copy(m_hbm.at[:, pl.ds(i * TB, TB), :], mv, sem)
        cin.start()
        cin.wait()
        nd = lax.rsqrt(jnp.maximum(cd_ref[:], 1.0))
        m = mv[0, :, :DO] + mv[1, :, :DO]
        o_ref[:] = m * nd[:, None] + b_ref[:][None, :]

    return pl.pallas_call(
        body,
        grid=(G,),
        in_specs=[
            pl.BlockSpec(memory_space=pl.ANY),
            pl.BlockSpec((TB,), lambda i: (i,)),
            pl.BlockSpec((DO,), lambda i: (0,)),
        ],
        out_specs=pl.BlockSpec((TB, DO), lambda i: (i, 0)),
        out_shape=jax.ShapeDtypeStruct((N, DO), jnp.float32),
        scratch_shapes=[
            pltpu.VMEM((NC, TB, DP), jnp.float32),
            pltpu.SemaphoreType.DMA,
        ],
    )(part2, cd, b2)


def kernel(features, edge_index, W1, b1, W2, b2):
    N, DIN = features.shape
    E = edge_index.shape[1]
    DH = W1.shape[1]
    DO = W2.shape[1]

    rows = -(-E // IDXW)
    K = -(-rows // (NC * NS * 36)) * 36
    R = K * NC * NS
    Epad = R * IDXW
    NP = -(-(N + 1) // TB) * TB
    DW = 16

    pad_idx = N + (jnp.arange(Epad - E, dtype=jnp.int32) % 128)
    idxp = jnp.concatenate(
        [edge_index, jnp.broadcast_to(pad_idx, (2, Epad - E))], axis=1
    )
    idx2 = idxp.reshape(2, R, IDXW)
    featsp = jnp.zeros((NP, DW), features.dtype).at[:N, :DIN].set(features)
    W1p = jnp.zeros((DW, DH), W1.dtype).at[:DIN, :].set(W1)

    cnts = _make_deg(NP, R)(idx2)
    xs = _prep(cnts[0], featsp)
    part1 = _make_agg(NP, R, K, DW, 6, 2)(idx2, xs, jnp.zeros((NP, DW), jnp.float32))
    table2 = _mid(part1, cnts[0], cnts[1], W1p, b1, W2)
    part2 = _make_agg(NP, R, K, 8, 9, 2)(idx2, table2, jnp.zeros((NP, 8), jnp.float32))
    return _fin(part2, cnts[1], b2, N)


# speedup vs baseline: 39.8150x; 1.0891x over previous
import functools

import numpy as np

import jax
import jax.numpy as jnp
from jax import lax
from jax.experimental import pallas as pl
from jax.experimental.pallas import tpu as pltpu
from jax.experimental.pallas import tpu_sc as plsc

NC = 2
NS = 16
LANES = 16
IDXW = 128
CHUNK = 8
TB = 7168


def _make_deg(NP, R):
    K2 = R // NS
    DCH = 12
    NB = 2
    nchunks = K2 // DCH
    G = nchunks // NB
    assert nchunks % NB == 0
    ZT = NP // NS
    mesh = plsc.VectorSubcoreMesh(core_axis_name="c", subcore_axis_name="s")

    @functools.partial(
        pl.kernel,
        out_type=jax.ShapeDtypeStruct((NC, NP), jnp.float32),
        mesh=mesh,
        compiler_params=pltpu.CompilerParams(use_tc_tiling_on_sc=False),
        scratch_types=[
            pltpu.VMEM_SHARED((NP,), jnp.float32),
            pltpu.VMEM((NB, DCH, IDXW), jnp.int32),
            pltpu.VMEM((IDXW,), jnp.float32),
            pltpu.VMEM((ZT,), jnp.float32),
            pltpu.SemaphoreType.DMA,
            pltpu.SemaphoreType.DMA,
            pltpu.SemaphoreType.DMA,
            pltpu.SemaphoreType.DMA,
        ],
    )
    def deg(idx_hbm, out_hbm, acc, sidx, ones, zbuf, sem0, sem1, isem0, isem1):
        c = lax.axis_index("c")
        s = lax.axis_index("s")
        sems = [sem0, sem1]
        isems = [isem0, isem1]

        def fill_ones(i, carry):
            ones[pl.ds(i * LANES, LANES)] = jnp.ones((LANES,), jnp.float32)
            return carry

        lax.fori_loop(0, IDXW // LANES, fill_ones, 0)

        def fill_z(i, carry):
            zbuf[pl.ds(i * LANES, LANES)] = jnp.zeros((LANES,), jnp.float32)
            return carry

        lax.fori_loop(0, ZT // LANES, fill_z, 0)
        pltpu.sync_copy(zbuf, acc.at[pl.ds(s * ZT, ZT)])
        plsc.subcore_barrier()

        base = s * K2
        icps = [
            pltpu.async_copy(
                idx_hbm.at[c, pl.ds(base + b * DCH, DCH), :], sidx.at[b],
                isems[b],
            )
            for b in range(NB)
        ]

        def group(g, carry):
            cps = []
            for b in range(NB):
                icps[b].wait()
                cps.append([
                    pltpu.async_copy(
                        ones, acc.at[sidx.at[b, j]], sems[b], add=True
                    )
                    for j in range(DCH)
                ])
            for b in range(NB):
                for cp in cps[b]:
                    cp.wait()
                nxt = jnp.minimum((g + 1) * NB + b, nchunks - 1)
                icps[b] = pltpu.async_copy(
                    idx_hbm.at[c, pl.ds(base + nxt * DCH, DCH), :], sidx.at[b],
                    isems[b],
                )
            return carry

        lax.fori_loop(0, G, group, 0)
        for b in range(NB):
            pltpu.make_async_copy(
                idx_hbm.at[c, pl.ds(0, DCH), :], sidx.at[b], isems[b]
            ).wait()
        plsc.subcore_barrier()
        pltpu.sync_copy(acc.at[pl.ds(s * ZT, ZT)], out_hbm.at[c, pl.ds(s * ZT, ZT)])

    return deg


def _make_agg(NP, R, K, D, CH, NB, DOUT=None):
    DOUT = D if DOUT is None else DOUT
    nchunks = K // CH
    G = nchunks // NB
    assert nchunks % NB == 0
    ZT = NP // NS
    mesh = plsc.VectorSubcoreMesh(core_axis_name="c", subcore_axis_name="s")

    @functools.partial(
        pl.kernel,
        out_type=jax.ShapeDtypeStruct((NC, NP, DOUT), jnp.float32),
        mesh=mesh,
        compiler_params=pltpu.CompilerParams(use_tc_tiling_on_sc=False),
        scratch_types=[
            pltpu.VMEM_SHARED((NP, D), jnp.float32),
            pltpu.VMEM((NB, 2, CH, IDXW), jnp.int32),
            pltpu.VMEM((NB, CH, IDXW, D), jnp.float32),
            pltpu.SemaphoreType.DMA,
            pltpu.SemaphoreType.DMA,
            pltpu.SemaphoreType.DMA,
            pltpu.SemaphoreType.DMA,
            pltpu.SemaphoreType.DMA,
            pltpu.SemaphoreType.DMA,
        ],
    )
    def agg(idx_hbm, table_hbm, zeros_hbm, out_hbm, acc, idxb, rows,
            gsem0, gsem1, ssem0, ssem1, isem0, isem1):
        c = lax.axis_index("c")
        s = lax.axis_index("s")
        wid = c * NS + s
        gsems = [gsem0, gsem1]
        ssems = [ssem0, ssem1]
        isems = [isem0, isem1]

        pltpu.sync_copy(zeros_hbm.at[pl.ds(s * ZT, ZT), :],
                        acc.at[pl.ds(s * ZT, ZT), :])
        plsc.subcore_barrier()

        base = wid * K

        def fire_idx(b, ci):
            return pltpu.async_copy(
                idx_hbm.at[:, pl.ds(base + ci * CH, CH), :], idxb.at[b],
                isems[b],
            )

        def fire_gathers(b):
            return [
                pltpu.async_copy(
                    table_hbm.at[idxb.at[b, 0, j]], rows.at[b, j], gsems[b]
                )
                for j in range(CH)
            ]

        icps = [fire_idx(b, b) for b in range(NB)]
        gcps = []
        for b in range(NB):
            icps[b].wait()
            gcps.append(fire_gathers(b))

        def group(g, carry):
            scps = []
            for b in range(NB):
                for cp in gcps[b]:
                    cp.wait()
                scps.append([
                    pltpu.async_copy(
                        rows.at[b, j], acc.at[idxb.at[b, 1, j]], ssems[b],
                        add=True,
                    )
                    for j in range(CH)
                ])
            for b in range(NB):
                for cp in scps[b]:
                    cp.wait()
                nxt = jnp.minimum((g + 1) * NB + b, nchunks - 1)
                icps[b] = fire_idx(b, nxt)
            for b in range(NB):
                icps[b].wait()
                gcps[b] = fire_gathers(b)
            return carry

        lax.fori_loop(0, G, group, 0)
        for b in range(NB):
            for j in range(CH):
                pltpu.make_async_copy(
                    table_hbm.at[idxb.at[b, 0, j]], rows.at[b, j], gsems[b]
                ).wait()
        plsc.subcore_barrier()
        pltpu.sync_copy(
            acc.at[pl.ds(s * ZT, ZT), pl.ds(0, DOUT)],
            out_hbm.at[c, pl.ds(s * ZT, ZT), :],
        )

    return agg


def _prep(cs, featsp):
    NP, DW = featsp.shape
    G = NP // TB

    def body(cs_ref, x_ref, t_ref):
        ns = lax.rsqrt(jnp.maximum(cs_ref[:], 1.0))
        t_ref[:] = x_ref[:] * ns[:, None]

    return pl.pallas_call(
        body,
        grid=(G,),
        in_specs=[
            pl.BlockSpec((TB,), lambda i: (i,)),
            pl.BlockSpec((TB, DW), lambda i: (i, 0)),
        ],
        out_specs=pl.BlockSpec((TB, DW), lambda i: (i, 0)),
        out_shape=jax.ShapeDtypeStruct((NP, DW), jnp.float32),
    )(cs, featsp)


def _mid(part1, cs, cd, W1p, b1, W2):
    _, NP, DW = part1.shape
    DO = W2.shape[1]
    G = NP // TB

    def body(m_ref, cs_ref, cd_ref, w1_ref, b_ref, w2_ref, t_ref):
        ns = lax.rsqrt(jnp.maximum(cs_ref[:], 1.0))
        nd = lax.rsqrt(jnp.maximum(cd_ref[:], 1.0))
        m = m_ref[0] + m_ref[1]
        h = jnp.dot(m, w1_ref[:], preferred_element_type=jnp.float32)
        x1 = jnp.maximum(h * nd[:, None] + b_ref[:][None, :], 0.0)
        y = jnp.dot(
            x1 * ns[:, None], w2_ref[:], preferred_element_type=jnp.float32
        )
        t_ref[:] = jnp.concatenate(
            [y, jnp.zeros((y.shape[0], 8 - DO), jnp.float32)], axis=1
        )

    return pl.pallas_call(
        body,
        grid=(G,),
        in_specs=[
            pl.BlockSpec((NC, TB, DW), lambda i: (0, i, 0)),
            pl.BlockSpec((TB,), lambda i: (i,)),
            pl.BlockSpec((TB,), lambda i: (i,)),
            pl.BlockSpec((DW, DW), lambda i: (0, 0)),
            pl.BlockSpec((DW,), lambda i: (0,)),
            pl.BlockSpec((DW, DO), lambda i: (0, 0)),
        ],
        out_specs=pl.BlockSpec((TB, 8), lambda i: (i, 0)),
        out_shape=jax.ShapeDtypeStruct((NP, 8), jnp.float32),
    )(part1, cs, cd, W1p, b1, W2)


def _fin(part2, cd, b2, N):
    _, NP, DP = part2.shape
    DO = b2.shape[0]
    G = -(-N // TB)

    def body(m_ref, cd_ref, b_ref, o_ref):
        nd = lax.rsqrt(jnp.maximum(cd_ref[:], 1.0))
        m = m_ref[0, :, :DO] + m_ref[1, :, :DO]
        o_ref[:] = m * nd[:, None] + b_ref[:][None, :]

    return pl.pallas_call(
        body,
        grid=(G,),
        in_specs=[
            pl.BlockSpec((NC, TB, DP), lambda i: (0, i, 0)),
            pl.BlockSpec((TB,), lambda i: (i,)),
            pl.BlockSpec((DO,), lambda i: (0,)),
        ],
        out_specs=pl.BlockSpec((TB, DO), lambda i: (i, 0)),
        out_shape=jax.ShapeDtypeStruct((N, DO), jnp.float32),
    )(part2, cd, b2)


def kernel(features, edge_index, W1, b1, W2, b2):
    N, DIN = features.shape
    E = edge_index.shape[1]
    DH = W1.shape[1]
    DO = W2.shape[1]

    rows = -(-E // IDXW)
    K = -(-rows // (NC * NS * 36)) * 36
    R = K * NC * NS
    Epad = R * IDXW
    NP = -(-(N + 1) // TB) * TB
    DW = 16

    pad_np = np.broadcast_to(
        (N + (np.arange(Epad - E) % 128)).astype(np.int32), (2, Epad - E)
    )
    idxp = jnp.concatenate([edge_index, jnp.asarray(pad_np)], axis=1)
    idx2 = idxp.reshape(2, R, IDXW)
    featsp = jnp.zeros((NP, DW), features.dtype).at[:N, :DIN].set(features)
    W1p = jnp.zeros((DW, DH), W1.dtype).at[:DIN, :].set(W1)

    cnts = _make_deg(NP, R)(idx2)
    xs = _prep(cnts[0], featsp)
    part1 = _make_agg(NP, R, K, DW, 6, 2)(idx2, xs, jnp.zeros((NP, DW), jnp.float32))
    table2 = _mid(part1, cnts[0], cnts[1], W1p, b1, W2)
    part2 = _make_agg(NP, R, K, 8, 12, 2)(
        idx2, table2, jnp.zeros((NP, 8), jnp.float32)
    )
    return _fin(part2, cnts[1], b2, N)
